# Initial kernel scaffold; baseline (speedup 1.0000x reference)
import jax
import jax.numpy as jnp
from jax.experimental import pallas as pl


def kernel(x, edge_index, W1, b1, W2, b2):
    raise NotImplementedError("write your pallas kernel here")


# speedup vs baseline: 26.2457x; 26.2457x over previous
import functools

import jax
import jax.numpy as jnp
from jax import lax
from jax.experimental import pallas as pl
from jax.experimental.pallas import tpu as pltpu
from jax.experimental.pallas import tpu_sc as plsc

N = 50000
NP = 50176
D_IN = 64
HIDDEN = 256
HALF = 32
NC = 2
NS = 16
CHUNK = 128
GROUP = 8
ACC_ROWS = 50304
ZSTRIPE = ACC_ROWS // NS
ZCHUNK = 393
OUT_STRIPE = NP // NS
OCHUNK = 392
TRASH0 = NP
N_TRASH = ACC_ROWS - NP

_mesh = plsc.VectorSubcoreMesh(core_axis_name="c", subcore_axis_name="s",
                               num_cores=NC, num_subcores=NS)


def _fill_zeros(buf, nrows, ncols):
    z = jnp.zeros((16,), jnp.float32)

    @pl.loop(0, nrows)
    def _(i):
        for c in range(ncols // 16):
            buf[i, pl.ds(c * 16, 16)] = z


def _zero_acc(acc, tmp, sid):
    for t in range(NS):
        @pl.when(sid == t)
        def _():
            for k in range(ZSTRIPE // ZCHUNK):
                pltpu.sync_copy(
                    tmp, acc.at[pl.ds(t * ZSTRIPE + k * ZCHUNK, ZCHUNK)])


def _copy_out_stripe(acc, out, tmp, sid):
    for t in range(NS):
        @pl.when(sid == t)
        def _():
            for k in range(OUT_STRIPE // OCHUNK):
                off = t * OUT_STRIPE + k * OCHUNK
                pltpu.sync_copy(acc.at[pl.ds(off, OCHUNK)],
                                tmp.at[pl.ds(0, OCHUNK)])
                pltpu.sync_copy(tmp.at[pl.ds(0, OCHUNK)],
                                out.at[pl.ds(off, OCHUNK)])


def _make_hist(rows):
    per_worker = rows // (NC * NS)
    ngroups = per_worker // GROUP

    @functools.partial(
        pl.kernel,
        out_type=(jax.ShapeDtypeStruct((NP, 16), jnp.float32),
                  jax.ShapeDtypeStruct((NP, 16), jnp.float32)),
        mesh=_mesh,
        compiler_params=pltpu.CompilerParams(use_tc_tiling_on_sc=False),
        scratch_types=[
            pltpu.VMEM_SHARED((ACC_ROWS, 16), jnp.float32),
            pltpu.VMEM((GROUP, CHUNK), jnp.int32),
            pltpu.VMEM((CHUNK, 16), jnp.float32),
            pltpu.VMEM((ZCHUNK, 16), jnp.float32),
        ],
    )
    def hist(dst2, out0, out1, acc, dbuf, ones, tmp):
        cid = lax.axis_index("c")
        sid = lax.axis_index("s")
        e0 = jnp.where(lax.iota(jnp.int32, 16) == 0, 1.0, 0.0)

        @pl.loop(0, CHUNK)
        def _(i):
            ones[i, pl.ds(0, 16)] = e0

        _fill_zeros(tmp, ZCHUNK, 16)
        _zero_acc(acc, tmp, sid)
        plsc.subcore_barrier()

        worker = cid * NS + sid

        @pl.loop(0, ngroups)
        def _(g):
            base = pl.multiple_of(worker * per_worker + g * GROUP, 8)
            pltpu.sync_copy(dst2.at[pl.ds(base, GROUP)], dbuf)
            for j in range(GROUP):
                pltpu.sync_copy(ones, acc.at[dbuf.at[j]], add=True)

        plsc.subcore_barrier()

        @pl.when(cid == 0)
        def _():
            _copy_out_stripe(acc, out0, tmp, sid)

        @pl.when(cid == 1)
        def _():
            _copy_out_stripe(acc, out1, tmp, sid)

    return hist


def _make_agg(rows):
    per_tile = rows // NS
    ngroups = per_tile // GROUP

    @functools.partial(
        pl.kernel,
        out_type=(jax.ShapeDtypeStruct((NP, HALF), jnp.float32),
                  jax.ShapeDtypeStruct((NP, HALF), jnp.float32)),
        mesh=_mesh,
        compiler_params=pltpu.CompilerParams(use_tc_tiling_on_sc=False),
        scratch_types=[
            pltpu.VMEM_SHARED((ACC_ROWS, HALF), jnp.float32),
            pltpu.VMEM((GROUP, CHUNK), jnp.int32),
            pltpu.VMEM((GROUP, CHUNK), jnp.int32),
            pltpu.VMEM((CHUNK, HALF), jnp.float32),
            pltpu.VMEM((CHUNK, HALF), jnp.float32),
            pltpu.VMEM((ZCHUNK, HALF), jnp.float32),
            pltpu.SemaphoreType.DMA,
            pltpu.SemaphoreType.DMA,
        ],
    )
    def agg(tab0, tab1, src2, dst2, out0, out1, acc, sbuf, dbuf, r0, r1,
            tmp, sem0, sem1):
        cid = lax.axis_index("c")
        sid = lax.axis_index("s")

        _fill_zeros(tmp, ZCHUNK, HALF)
        _zero_acc(acc, tmp, sid)
        plsc.subcore_barrier()

        bufs = (r0, r1)
        sems = (sem0, sem1)

        def pipeline(table, out):
            @pl.loop(0, ngroups)
            def _(g):
                base = pl.multiple_of((sid * ngroups + g) * GROUP, 8)
                pltpu.sync_copy(src2.at[pl.ds(base, GROUP)], sbuf)
                pltpu.sync_copy(dst2.at[pl.ds(base, GROUP)], dbuf)
                copies = {}
                copies[0] = pltpu.async_copy(
                    table.at[sbuf.at[0]], bufs[0], sems[0])
                for j in range(GROUP):
                    if j + 1 < GROUP:
                        copies[j + 1] = pltpu.async_copy(
                            table.at[sbuf.at[j + 1]], bufs[(j + 1) % 2],
                            sems[(j + 1) % 2])
                    copies[j].wait()
                    pltpu.sync_copy(bufs[j % 2], acc.at[dbuf.at[j]], add=True)

            plsc.subcore_barrier()
            _copy_out_stripe(acc, out, tmp, sid)

        @pl.when(cid == 0)
        def _():
            pipeline(tab0, out0)

        @pl.when(cid == 1)
        def _():
            pipeline(tab1, out1)

    return agg


_BR = 3136
_NBLK = NP // _BR


def _dinv_from(d0, d1):
    return lax.rsqrt(d0[:, 0] + d1[:, 0] + 1.0)


def _prep_body(x_ref, d0_ref, d1_ref, xs0_ref, xs1_ref):
    dinv = _dinv_from(d0_ref[...], d1_ref[...])
    xs = x_ref[...] * dinv[:, None]
    xs0_ref[...] = xs[:, :HALF]
    xs1_ref[...] = xs[:, HALF:]


def _mid_body(s0_ref, s1_ref, xs0_ref, xs1_ref, d0_ref, d1_ref, w1_ref,
              b1_ref, w2_ref, ts0_ref, ts1_ref):
    dinv = _dinv_from(d0_ref[...], d1_ref[...])[:, None]
    a0 = (s0_ref[...] + xs0_ref[...]) * dinv
    a1 = (s1_ref[...] + xs1_ref[...]) * dinv
    w1 = w1_ref[...]
    h = (jnp.dot(a0, w1[:HALF], preferred_element_type=jnp.float32)
         + jnp.dot(a1, w1[HALF:], preferred_element_type=jnp.float32))
    h = jnp.maximum(h + b1_ref[...], 0.0)
    t = jnp.dot(h, w2_ref[...], preferred_element_type=jnp.float32)
    ts = t * dinv
    ts0_ref[...] = ts[:, :HALF]
    ts1_ref[...] = ts[:, HALF:]


def _final_body(t0_ref, t1_ref, ts0_ref, ts1_ref, d0_ref, d1_ref, b2_ref,
                out_ref):
    dinv = _dinv_from(d0_ref[...], d1_ref[...])[:, None]
    b2 = b2_ref[...]
    o0 = (t0_ref[...] + ts0_ref[...]) * dinv + b2[:, :HALF]
    o1 = (t1_ref[...] + ts1_ref[...]) * dinv + b2[:, HALF:]
    out_ref[...] = jnp.maximum(jnp.concatenate([o0, o1], axis=1), 0.0)


def _row_spec(cols):
    return pl.BlockSpec((_BR, cols), lambda i: (i, 0))


def _full_spec(shape):
    return pl.BlockSpec(shape, lambda i: tuple(0 for _ in shape))


_prep_call = pl.pallas_call(
    _prep_body,
    grid=(_NBLK,),
    in_specs=[_row_spec(D_IN), _row_spec(16), _row_spec(16)],
    out_specs=[_row_spec(HALF), _row_spec(HALF)],
    out_shape=[jax.ShapeDtypeStruct((NP, HALF), jnp.float32)] * 2,
)

_mid_call = pl.pallas_call(
    _mid_body,
    grid=(_NBLK,),
    in_specs=[_row_spec(HALF), _row_spec(HALF), _row_spec(HALF),
              _row_spec(HALF), _row_spec(16), _row_spec(16),
              _full_spec((D_IN, HIDDEN)), _full_spec((1, HIDDEN)),
              _full_spec((HIDDEN, D_IN))],
    out_specs=[_row_spec(HALF), _row_spec(HALF)],
    out_shape=[jax.ShapeDtypeStruct((NP, HALF), jnp.float32)] * 2,
)

_final_call = pl.pallas_call(
    _final_body,
    grid=(_NBLK,),
    in_specs=[_row_spec(HALF), _row_spec(HALF), _row_spec(HALF),
              _row_spec(HALF), _row_spec(16), _row_spec(16),
              _full_spec((1, D_IN))],
    out_specs=_row_spec(D_IN),
    out_shape=jax.ShapeDtypeStruct((NP, D_IN), jnp.float32),
)


@jax.jit
def kernel(x, edge_index, W1, b1, W2, b2):
    E = edge_index.shape[1]
    unit = NC * NS * GROUP * CHUNK
    E_pad = ((E + unit - 1) // unit) * unit
    rows = E_pad // CHUNK
    pad = E_pad - E

    src = edge_index[0]
    dst = edge_index[1]
    pad_ar = jnp.arange(pad, dtype=jnp.int32)
    srcp = jnp.concatenate([src, pad_ar % N]).reshape(rows, CHUNK)
    dstp = jnp.concatenate([dst, TRASH0 + pad_ar % N_TRASH]).reshape(rows, CHUNK)
    xp = jnp.pad(x, ((0, NP - N), (0, 0)))

    d0, d1 = _make_hist(rows)(dstp)
    xs0, xs1 = _prep_call(xp, d0, d1)
    agg = _make_agg(rows)
    s0, s1 = agg(xs0, xs1, srcp, dstp)
    ts0, ts1 = _mid_call(s0, s1, xs0, xs1, d0, d1, W1, b1.reshape(1, -1), W2)
    t0, t1 = agg(ts0, ts1, srcp, dstp)
    return _final_call(t0, t1, ts0, ts1, d0, d1, b2.reshape(1, -1))[:N]


# speedup vs baseline: 28.9905x; 1.1046x over previous
import functools

import jax
import jax.numpy as jnp
from jax import lax
from jax.experimental import pallas as pl
from jax.experimental.pallas import tpu as pltpu
from jax.experimental.pallas import tpu_sc as plsc

N = 50000
NP = 50176
D_IN = 64
HIDDEN = 256
HALF = 32
NC = 2
NS = 16
CHUNK = 128
GROUP = 8
ACC_ROWS = 50304
ZSTRIPE = ACC_ROWS // NS
ZCHUNK = 262
OUT_STRIPE = NP // NS
OCHUNK = 224
NBUF = 4
TRASH0 = NP
N_TRASH = ACC_ROWS - NP

_mesh = plsc.VectorSubcoreMesh(core_axis_name="c", subcore_axis_name="s",
                               num_cores=NC, num_subcores=NS)


def _fill_zeros(buf, nrows, ncols):
    z = jnp.zeros((16,), jnp.float32)

    @pl.loop(0, nrows)
    def _(i):
        for c in range(ncols // 16):
            buf[i, pl.ds(c * 16, 16)] = z


def _zero_acc(acc, tmp, sid):
    for t in range(NS):
        @pl.when(sid == t)
        def _():
            for k in range(ZSTRIPE // ZCHUNK):
                pltpu.sync_copy(
                    tmp, acc.at[pl.ds(t * ZSTRIPE + k * ZCHUNK, ZCHUNK)])


def _copy_out_stripe(acc, out, tmp, sid):
    for t in range(NS):
        @pl.when(sid == t)
        def _():
            for k in range(OUT_STRIPE // OCHUNK):
                off = t * OUT_STRIPE + k * OCHUNK
                pltpu.sync_copy(acc.at[pl.ds(off, OCHUNK)],
                                tmp.at[pl.ds(0, OCHUNK)])
                pltpu.sync_copy(tmp.at[pl.ds(0, OCHUNK)],
                                out.at[pl.ds(off, OCHUNK)])


def _make_hist(rows):
    per_worker = rows // (NC * NS)
    ngroups = per_worker // GROUP

    @functools.partial(
        pl.kernel,
        out_type=(jax.ShapeDtypeStruct((NP, 16), jnp.float32),
                  jax.ShapeDtypeStruct((NP, 16), jnp.float32)),
        mesh=_mesh,
        compiler_params=pltpu.CompilerParams(use_tc_tiling_on_sc=False),
        scratch_types=[
            pltpu.VMEM_SHARED((ACC_ROWS, 16), jnp.float32),
            pltpu.VMEM((GROUP, CHUNK), jnp.int32),
            pltpu.VMEM((CHUNK, 16), jnp.float32),
            pltpu.VMEM((ZCHUNK, 16), jnp.float32),
        ],
    )
    def hist(dst2, out0, out1, acc, dbuf, ones, tmp):
        cid = lax.axis_index("c")
        sid = lax.axis_index("s")
        e0 = jnp.where(lax.iota(jnp.int32, 16) == 0, 1.0, 0.0)

        @pl.loop(0, CHUNK)
        def _(i):
            ones[i, pl.ds(0, 16)] = e0

        _fill_zeros(tmp, ZCHUNK, 16)
        _zero_acc(acc, tmp, sid)
        plsc.subcore_barrier()

        worker = cid * NS + sid

        @pl.loop(0, ngroups)
        def _(g):
            base = pl.multiple_of(worker * per_worker + g * GROUP, 8)
            pltpu.sync_copy(dst2.at[pl.ds(base, GROUP)], dbuf)
            for j in range(GROUP):
                pltpu.sync_copy(ones, acc.at[dbuf.at[j]], add=True)

        plsc.subcore_barrier()

        @pl.when(cid == 0)
        def _():
            _copy_out_stripe(acc, out0, tmp, sid)

        @pl.when(cid == 1)
        def _():
            _copy_out_stripe(acc, out1, tmp, sid)

    return hist


def _make_agg(rows):
    per_tile = rows // NS
    ngroups = per_tile // GROUP

    @functools.partial(
        pl.kernel,
        out_type=(jax.ShapeDtypeStruct((NP, HALF), jnp.float32),
                  jax.ShapeDtypeStruct((NP, HALF), jnp.float32)),
        mesh=_mesh,
        compiler_params=pltpu.CompilerParams(use_tc_tiling_on_sc=False),
        scratch_types=[
            pltpu.VMEM_SHARED((ACC_ROWS, HALF), jnp.float32),
            pltpu.VMEM((GROUP, CHUNK), jnp.int32),
            pltpu.VMEM((GROUP, CHUNK), jnp.int32),
            pltpu.VMEM((CHUNK, HALF), jnp.float32),
            pltpu.VMEM((CHUNK, HALF), jnp.float32),
            pltpu.VMEM((CHUNK, HALF), jnp.float32),
            pltpu.VMEM((CHUNK, HALF), jnp.float32),
            pltpu.VMEM((ZCHUNK, HALF), jnp.float32),
            pltpu.SemaphoreType.DMA,
            pltpu.SemaphoreType.DMA,
            pltpu.SemaphoreType.DMA,
            pltpu.SemaphoreType.DMA,
            pltpu.SemaphoreType.DMA,
            pltpu.SemaphoreType.DMA,
            pltpu.SemaphoreType.DMA,
            pltpu.SemaphoreType.DMA,
        ],
    )
    def agg(tab0, tab1, src2, dst2, out0, out1, acc, sbuf, dbuf,
            r0, r1, r2, r3, tmp, g0, g1, g2, g3, s0, s1, s2, s3):
        cid = lax.axis_index("c")
        sid = lax.axis_index("s")

        _fill_zeros(tmp, ZCHUNK, HALF)
        _zero_acc(acc, tmp, sid)
        plsc.subcore_barrier()

        bufs = (r0, r1, r2, r3)
        gsems = (g0, g1, g2, g3)
        ssems = (s0, s1, s2, s3)

        def pipeline(table, out):
            @pl.loop(0, ngroups)
            def _(g):
                base = pl.multiple_of((sid * ngroups + g) * GROUP, 8)
                pltpu.sync_copy(src2.at[pl.ds(base, GROUP)], sbuf)
                pltpu.sync_copy(dst2.at[pl.ds(base, GROUP)], dbuf)
                gath, scat = {}, {}
                for j in range(2):
                    gath[j] = pltpu.async_copy(
                        table.at[sbuf.at[j]], bufs[j % NBUF], gsems[j % NBUF])
                for j in range(GROUP):
                    nxt = j + 2
                    if nxt < GROUP:
                        if nxt >= NBUF:
                            scat[nxt - NBUF].wait()
                        gath[nxt] = pltpu.async_copy(
                            table.at[sbuf.at[nxt]], bufs[nxt % NBUF],
                            gsems[nxt % NBUF])
                    gath[j].wait()
                    scat[j] = pltpu.async_copy(
                        bufs[j % NBUF], acc.at[dbuf.at[j]], ssems[j % NBUF],
                        add=True)
                for j in range(GROUP - NBUF, GROUP):
                    scat[j].wait()

            plsc.subcore_barrier()
            _copy_out_stripe(acc, out, tmp, sid)

        @pl.when(cid == 0)
        def _():
            pipeline(tab0, out0)

        @pl.when(cid == 1)
        def _():
            pipeline(tab1, out1)

    return agg


_BR = 3136
_NBLK = NP // _BR


def _dinv_from(d0, d1):
    return lax.rsqrt(d0[:, 0] + d1[:, 0] + 1.0)


def _prep_body(x_ref, d0_ref, d1_ref, xs0_ref, xs1_ref):
    dinv = _dinv_from(d0_ref[...], d1_ref[...])
    xs = x_ref[...] * dinv[:, None]
    xs0_ref[...] = xs[:, :HALF]
    xs1_ref[...] = xs[:, HALF:]


def _mid_body(s0_ref, s1_ref, xs0_ref, xs1_ref, d0_ref, d1_ref, w1_ref,
              b1_ref, w2_ref, ts0_ref, ts1_ref):
    dinv = _dinv_from(d0_ref[...], d1_ref[...])[:, None]
    a0 = (s0_ref[...] + xs0_ref[...]) * dinv
    a1 = (s1_ref[...] + xs1_ref[...]) * dinv
    w1 = w1_ref[...]
    h = (jnp.dot(a0, w1[:HALF], preferred_element_type=jnp.float32)
         + jnp.dot(a1, w1[HALF:], preferred_element_type=jnp.float32))
    h = jnp.maximum(h + b1_ref[...], 0.0)
    t = jnp.dot(h, w2_ref[...], preferred_element_type=jnp.float32)
    ts = t * dinv
    ts0_ref[...] = ts[:, :HALF]
    ts1_ref[...] = ts[:, HALF:]


def _final_body(t0_ref, t1_ref, ts0_ref, ts1_ref, d0_ref, d1_ref, b2_ref,
                out_ref):
    dinv = _dinv_from(d0_ref[...], d1_ref[...])[:, None]
    b2 = b2_ref[...]
    o0 = (t0_ref[...] + ts0_ref[...]) * dinv + b2[:, :HALF]
    o1 = (t1_ref[...] + ts1_ref[...]) * dinv + b2[:, HALF:]
    out_ref[...] = jnp.maximum(jnp.concatenate([o0, o1], axis=1), 0.0)


def _row_spec(cols):
    return pl.BlockSpec((_BR, cols), lambda i: (i, 0))


def _full_spec(shape):
    return pl.BlockSpec(shape, lambda i: tuple(0 for _ in shape))


_prep_call = pl.pallas_call(
    _prep_body,
    grid=(_NBLK,),
    in_specs=[_row_spec(D_IN), _row_spec(16), _row_spec(16)],
    out_specs=[_row_spec(HALF), _row_spec(HALF)],
    out_shape=[jax.ShapeDtypeStruct((NP, HALF), jnp.float32)] * 2,
)

_mid_call = pl.pallas_call(
    _mid_body,
    grid=(_NBLK,),
    in_specs=[_row_spec(HALF), _row_spec(HALF), _row_spec(HALF),
              _row_spec(HALF), _row_spec(16), _row_spec(16),
              _full_spec((D_IN, HIDDEN)), _full_spec((1, HIDDEN)),
              _full_spec((HIDDEN, D_IN))],
    out_specs=[_row_spec(HALF), _row_spec(HALF)],
    out_shape=[jax.ShapeDtypeStruct((NP, HALF), jnp.float32)] * 2,
)

_final_call = pl.pallas_call(
    _final_body,
    grid=(_NBLK,),
    in_specs=[_row_spec(HALF), _row_spec(HALF), _row_spec(HALF),
              _row_spec(HALF), _row_spec(16), _row_spec(16),
              _full_spec((1, D_IN))],
    out_specs=_row_spec(D_IN),
    out_shape=jax.ShapeDtypeStruct((NP, D_IN), jnp.float32),
)


@jax.jit
def kernel(x, edge_index, W1, b1, W2, b2):
    E = edge_index.shape[1]
    unit = NC * NS * GROUP * CHUNK
    E_pad = ((E + unit - 1) // unit) * unit
    rows = E_pad // CHUNK
    pad = E_pad - E

    src = edge_index[0]
    dst = edge_index[1]
    pad_ar = jnp.arange(pad, dtype=jnp.int32)
    srcp = jnp.concatenate([src, pad_ar % N]).reshape(rows, CHUNK)
    dstp = jnp.concatenate([dst, TRASH0 + pad_ar % N_TRASH]).reshape(rows, CHUNK)
    xp = jnp.pad(x, ((0, NP - N), (0, 0)))

    d0, d1 = _make_hist(rows)(dstp)
    xs0, xs1 = _prep_call(xp, d0, d1)
    agg = _make_agg(rows)
    s0, s1 = agg(xs0, xs1, srcp, dstp)
    ts0, ts1 = _mid_call(s0, s1, xs0, xs1, d0, d1, W1, b1.reshape(1, -1), W2)
    t0, t1 = agg(ts0, ts1, srcp, dstp)
    return _final_call(t0, t1, ts0, ts1, d0, d1, b2.reshape(1, -1))[:N]


# speedup vs baseline: 31.9642x; 1.1026x over previous
import functools

import jax
import jax.numpy as jnp
from jax import lax
from jax.experimental import pallas as pl
from jax.experimental.pallas import tpu as pltpu
from jax.experimental.pallas import tpu_sc as plsc

N = 50000
NP = 50176
D_IN = 64
HIDDEN = 256
HALF = 32
NC = 2
NS = 16
CHUNK = 128
GROUP = 8
AGROUP = 16
ACC_ROWS = 50304
ZSTRIPE = ACC_ROWS // NS
ZCHUNK = 262
OUT_STRIPE = NP // NS
OCHUNK = 224
NBUF = 4
TRASH0 = NP
N_TRASH = ACC_ROWS - NP

_mesh = plsc.VectorSubcoreMesh(core_axis_name="c", subcore_axis_name="s",
                               num_cores=NC, num_subcores=NS)


def _fill_zeros(buf, nrows, ncols):
    z = jnp.zeros((16,), jnp.float32)

    @pl.loop(0, nrows)
    def _(i):
        for c in range(ncols // 16):
            buf[i, pl.ds(c * 16, 16)] = z


def _zero_acc(acc, tmp, sid):
    for t in range(NS):
        @pl.when(sid == t)
        def _():
            for k in range(ZSTRIPE // ZCHUNK):
                pltpu.sync_copy(
                    tmp, acc.at[pl.ds(t * ZSTRIPE + k * ZCHUNK, ZCHUNK)])


def _copy_out_stripe(acc, out, tmp, sid):
    for t in range(NS):
        @pl.when(sid == t)
        def _():
            for k in range(OUT_STRIPE // OCHUNK):
                off = t * OUT_STRIPE + k * OCHUNK
                pltpu.sync_copy(acc.at[pl.ds(off, OCHUNK)],
                                tmp.at[pl.ds(0, OCHUNK)])
                pltpu.sync_copy(tmp.at[pl.ds(0, OCHUNK)],
                                out.at[pl.ds(off, OCHUNK)])


def _make_hist(rows):
    per_worker = rows // (NC * NS)
    ngroups = per_worker // GROUP

    @functools.partial(
        pl.kernel,
        out_type=(jax.ShapeDtypeStruct((NP, 16), jnp.float32),
                  jax.ShapeDtypeStruct((NP, 16), jnp.float32)),
        mesh=_mesh,
        compiler_params=pltpu.CompilerParams(use_tc_tiling_on_sc=False),
        scratch_types=[
            pltpu.VMEM_SHARED((ACC_ROWS, 16), jnp.float32),
            pltpu.VMEM((GROUP, CHUNK), jnp.int32),
            pltpu.VMEM((CHUNK, 16), jnp.float32),
            pltpu.VMEM((ZCHUNK, 16), jnp.float32),
        ],
    )
    def hist(dst2, out0, out1, acc, dbuf, ones, tmp):
        cid = lax.axis_index("c")
        sid = lax.axis_index("s")
        e0 = jnp.where(lax.iota(jnp.int32, 16) == 0, 1.0, 0.0)

        @pl.loop(0, CHUNK)
        def _(i):
            ones[i, pl.ds(0, 16)] = e0

        _fill_zeros(tmp, ZCHUNK, 16)
        _zero_acc(acc, tmp, sid)
        plsc.subcore_barrier()

        worker = cid * NS + sid

        @pl.loop(0, ngroups)
        def _(g):
            base = pl.multiple_of(worker * per_worker + g * GROUP, 8)
            pltpu.sync_copy(dst2.at[pl.ds(base, GROUP)], dbuf)
            for j in range(GROUP):
                pltpu.sync_copy(ones, acc.at[dbuf.at[j]], add=True)

        plsc.subcore_barrier()

        @pl.when(cid == 0)
        def _():
            _copy_out_stripe(acc, out0, tmp, sid)

        @pl.when(cid == 1)
        def _():
            _copy_out_stripe(acc, out1, tmp, sid)

    return hist


def _make_agg(rows):
    per_tile = rows // NS
    ngroups = per_tile // AGROUP

    @functools.partial(
        pl.kernel,
        out_type=(jax.ShapeDtypeStruct((NP, HALF), jnp.float32),
                  jax.ShapeDtypeStruct((NP, HALF), jnp.float32)),
        mesh=_mesh,
        compiler_params=pltpu.CompilerParams(use_tc_tiling_on_sc=False),
        scratch_types=[
            pltpu.VMEM_SHARED((ACC_ROWS, HALF), jnp.float32),
            pltpu.VMEM((AGROUP, CHUNK), jnp.int32),
            pltpu.VMEM((AGROUP, CHUNK), jnp.int32),
            pltpu.VMEM((CHUNK, HALF), jnp.float32),
            pltpu.VMEM((CHUNK, HALF), jnp.float32),
            pltpu.VMEM((CHUNK, HALF), jnp.float32),
            pltpu.VMEM((CHUNK, HALF), jnp.float32),
            pltpu.VMEM((ZCHUNK, HALF), jnp.float32),
            pltpu.SemaphoreType.DMA,
            pltpu.SemaphoreType.DMA,
            pltpu.SemaphoreType.DMA,
            pltpu.SemaphoreType.DMA,
            pltpu.SemaphoreType.DMA,
            pltpu.SemaphoreType.DMA,
            pltpu.SemaphoreType.DMA,
            pltpu.SemaphoreType.DMA,
        ],
    )
    def agg(tab0, tab1, src2, dst2, out0, out1, acc, sbuf, dbuf,
            r0, r1, r2, r3, tmp, g0, g1, g2, g3, s0, s1, s2, s3):
        cid = lax.axis_index("c")
        sid = lax.axis_index("s")

        _fill_zeros(tmp, ZCHUNK, HALF)
        _zero_acc(acc, tmp, sid)
        plsc.subcore_barrier()

        bufs = (r0, r1, r2, r3)
        gsems = (g0, g1, g2, g3)
        ssems = (s0, s1, s2, s3)

        def pipeline(table, out):
            @pl.loop(0, ngroups)
            def _(g):
                base = pl.multiple_of((sid * ngroups + g) * AGROUP, 8)
                pltpu.sync_copy(src2.at[pl.ds(base, AGROUP)], sbuf)
                pltpu.sync_copy(dst2.at[pl.ds(base, AGROUP)], dbuf)
                gath, scat = {}, {}
                for j in range(2):
                    gath[j] = pltpu.async_copy(
                        table.at[sbuf.at[j]], bufs[j % NBUF], gsems[j % NBUF])
                for j in range(AGROUP):
                    nxt = j + 2
                    if nxt < AGROUP:
                        if nxt >= NBUF:
                            scat[nxt - NBUF].wait()
                        gath[nxt] = pltpu.async_copy(
                            table.at[sbuf.at[nxt]], bufs[nxt % NBUF],
                            gsems[nxt % NBUF])
                    gath[j].wait()
                    scat[j] = pltpu.async_copy(
                        bufs[j % NBUF], acc.at[dbuf.at[j]], ssems[j % NBUF],
                        add=True)
                for j in range(AGROUP - NBUF, AGROUP):
                    scat[j].wait()

            plsc.subcore_barrier()
            _copy_out_stripe(acc, out, tmp, sid)

        @pl.when(cid == 0)
        def _():
            pipeline(tab0, out0)

        @pl.when(cid == 1)
        def _():
            pipeline(tab1, out1)

    return agg


_BR = 3136
_NBLK = NP // _BR


def _dinv_from(d0, d1):
    return lax.rsqrt(d0[:, 0] + d1[:, 0] + 1.0)


def _prep_body(x_ref, d0_ref, d1_ref, xs0_ref, xs1_ref):
    dinv = _dinv_from(d0_ref[...], d1_ref[...])
    xs = x_ref[...] * dinv[:, None]
    xs0_ref[...] = xs[:, :HALF]
    xs1_ref[...] = xs[:, HALF:]


def _mid_body(s0_ref, s1_ref, xs0_ref, xs1_ref, d0_ref, d1_ref, w1_ref,
              b1_ref, w2_ref, ts0_ref, ts1_ref):
    dinv = _dinv_from(d0_ref[...], d1_ref[...])[:, None]
    a0 = (s0_ref[...] + xs0_ref[...]) * dinv
    a1 = (s1_ref[...] + xs1_ref[...]) * dinv
    w1 = w1_ref[...]
    h = (jnp.dot(a0, w1[:HALF], preferred_element_type=jnp.float32)
         + jnp.dot(a1, w1[HALF:], preferred_element_type=jnp.float32))
    h = jnp.maximum(h + b1_ref[...], 0.0)
    t = jnp.dot(h, w2_ref[...], preferred_element_type=jnp.float32)
    ts = t * dinv
    ts0_ref[...] = ts[:, :HALF]
    ts1_ref[...] = ts[:, HALF:]


def _final_body(t0_ref, t1_ref, ts0_ref, ts1_ref, d0_ref, d1_ref, b2_ref,
                out_ref):
    dinv = _dinv_from(d0_ref[...], d1_ref[...])[:, None]
    b2 = b2_ref[...]
    o0 = (t0_ref[...] + ts0_ref[...]) * dinv + b2[:, :HALF]
    o1 = (t1_ref[...] + ts1_ref[...]) * dinv + b2[:, HALF:]
    out_ref[...] = jnp.maximum(jnp.concatenate([o0, o1], axis=1), 0.0)


def _row_spec(cols):
    return pl.BlockSpec((_BR, cols), lambda i: (i, 0))


def _full_spec(shape):
    return pl.BlockSpec(shape, lambda i: tuple(0 for _ in shape))


_prep_call = pl.pallas_call(
    _prep_body,
    grid=(_NBLK,),
    in_specs=[_row_spec(D_IN), _row_spec(16), _row_spec(16)],
    out_specs=[_row_spec(HALF), _row_spec(HALF)],
    out_shape=[jax.ShapeDtypeStruct((NP, HALF), jnp.float32)] * 2,
)

_mid_call = pl.pallas_call(
    _mid_body,
    grid=(_NBLK,),
    in_specs=[_row_spec(HALF), _row_spec(HALF), _row_spec(HALF),
              _row_spec(HALF), _row_spec(16), _row_spec(16),
              _full_spec((D_IN, HIDDEN)), _full_spec((1, HIDDEN)),
              _full_spec((HIDDEN, D_IN))],
    out_specs=[_row_spec(HALF), _row_spec(HALF)],
    out_shape=[jax.ShapeDtypeStruct((NP, HALF), jnp.float32)] * 2,
)

_final_call = pl.pallas_call(
    _final_body,
    grid=(_NBLK,),
    in_specs=[_row_spec(HALF), _row_spec(HALF), _row_spec(HALF),
              _row_spec(HALF), _row_spec(16), _row_spec(16),
              _full_spec((1, D_IN))],
    out_specs=_row_spec(D_IN),
    out_shape=jax.ShapeDtypeStruct((NP, D_IN), jnp.float32),
)


@jax.jit
def kernel(x, edge_index, W1, b1, W2, b2):
    E = edge_index.shape[1]
    unit = NC * NS * GROUP * CHUNK
    E_pad = ((E + unit - 1) // unit) * unit
    rows = E_pad // CHUNK
    pad = E_pad - E

    src = edge_index[0]
    dst = edge_index[1]
    pad_ar = jnp.arange(pad, dtype=jnp.int32)
    srcp = jnp.concatenate([src, pad_ar % N]).reshape(rows, CHUNK)
    dstp = jnp.concatenate([dst, TRASH0 + pad_ar % N_TRASH]).reshape(rows, CHUNK)
    xp = jnp.pad(x, ((0, NP - N), (0, 0)))

    d0, d1 = _make_hist(rows)(dstp)
    xs0, xs1 = _prep_call(xp, d0, d1)
    agg = _make_agg(rows)
    s0, s1 = agg(xs0, xs1, srcp, dstp)
    ts0, ts1 = _mid_call(s0, s1, xs0, xs1, d0, d1, W1, b1.reshape(1, -1), W2)
    t0, t1 = agg(ts0, ts1, srcp, dstp)
    return _final_call(t0, t1, ts0, ts1, d0, d1, b2.reshape(1, -1))[:N]
